# Initial kernel scaffold; baseline (speedup 1.0000x reference)
import jax
import jax.numpy as jnp
from jax.experimental import pallas as pl


def kernel(x, quantiles, probs, initial_means, initial_stds):
    raise NotImplementedError("write your pallas kernel here")


# speedup vs baseline: 86.5171x; 86.5171x over previous
import functools

import jax
import jax.numpy as jnp
from jax import lax
from jax.experimental import pallas as pl
from jax.experimental.pallas import tpu as pltpu
from jax.experimental.pallas import tpu_sc as plsc

_K = 99
_PAD_K = 128
_EPS = 1e-05
_D = 26
_B = 16384
_N = _B * _D
_NW = 32
_CHUNK = _N // _NW
_PERIOD = 13
_GROUPS = _CHUNK // (16 * _PERIOD)


def _body(x_hbm, q_hbm, p_hbm, m_hbm, s_hbm, out_hbm,
          x_v, o_v, q_v, p_v, m_v, s_v, patq_v, patm_v, pats_v):
    wid = lax.axis_index("s") * 2 + lax.axis_index("c")
    base = wid * _CHUNK

    pltpu.sync_copy(x_hbm.at[pl.ds(base, _CHUNK)], x_v)
    pltpu.sync_copy(q_hbm, q_v)
    pltpu.sync_copy(p_hbm, p_v)
    pltpu.sync_copy(m_hbm, m_v)
    pltpu.sync_copy(s_hbm, s_v)

    for j in range(_PERIOD):
        lane = lax.iota(jnp.int32, 16) + (j * 16)
        dd = lane % _D
        patq_v[pl.ds(j * 16, 16)] = dd * _PAD_K
        patm_v[pl.ds(j * 16, 16)] = plsc.load_gather(m_v, [dd])
        pats_v[pl.ds(j * 16, 16)] = 1.0 / plsc.load_gather(s_v, [dd])

    def group(g, carry):
        for j in range(_PERIOD):
            off = g * (_PERIOD * 16) + (j * 16)
            xv = x_v[pl.ds(off, 16)]
            qb = patq_v[pl.ds(j * 16, 16)]
            mv = patm_v[pl.ds(j * 16, 16)]
            iv = pats_v[pl.ds(j * 16, 16)]
            xn = (xv - mv) * iv

            apos = qb
            for step in (64, 32, 16, 8, 4, 2, 1):
                qv = plsc.load_gather(q_v, [apos + (step - 1)])
                apos = jnp.where(qv < xn, apos + step, apos)
            idx = apos - qb

            left = jnp.maximum(idx - 1, 0)
            right = jnp.minimum(idx, _K - 1)
            ql = plsc.load_gather(q_v, [qb + left])
            qr = plsc.load_gather(q_v, [qb + right])
            pL = plsc.load_gather(p_v, [left])
            pR = plsc.load_gather(p_v, [right])

            res = pL + (xn - ql) * (pR - pL) / (qr - ql + _EPS)
            mlow = (idx == 0) & (xn < ql)
            mhigh = (idx == _K) & (xn > qr)
            ta = jnp.where(mlow, ql - xn, xn - qr)
            e = jnp.exp(-2.0 * jnp.maximum(ta, 0.0))
            th = (1.0 - e) / (1.0 + e)
            res = jnp.where(mlow, pL - pL * th, res)
            res = jnp.where(mhigh, pR + (1.0 - pR) * th, res)
            o_v[pl.ds(off, 16)] = res
        return carry

    lax.fori_loop(0, _GROUPS, group, 0)
    pltpu.sync_copy(o_v, out_hbm.at[pl.ds(base, _CHUNK)])


@jax.jit
def _qnorm(xf, qpad, ppad, mpad, spad):
    mesh = plsc.VectorSubcoreMesh(core_axis_name="c", subcore_axis_name="s")
    f = pl.kernel(
        _body,
        out_type=jax.ShapeDtypeStruct((_N,), jnp.float32),
        mesh=mesh,
        compiler_params=pltpu.CompilerParams(needs_layout_passes=False),
        scratch_types=[
            pltpu.VMEM((_CHUNK,), jnp.float32),
            pltpu.VMEM((_CHUNK,), jnp.float32),
            pltpu.VMEM((_D * _PAD_K,), jnp.float32),
            pltpu.VMEM((_PAD_K,), jnp.float32),
            pltpu.VMEM((32,), jnp.float32),
            pltpu.VMEM((32,), jnp.float32),
            pltpu.VMEM((16 * _PERIOD,), jnp.int32),
            pltpu.VMEM((16 * _PERIOD,), jnp.float32),
            pltpu.VMEM((16 * _PERIOD,), jnp.float32),
        ],
    )
    return f(xf, qpad, ppad, mpad, spad)


def kernel(x, quantiles, probs, initial_means, initial_stds):
    d, k = quantiles.shape
    qpad = jnp.concatenate(
        [quantiles, jnp.full((d, _PAD_K - k), jnp.inf, jnp.float32)], axis=1
    ).reshape(-1)
    ppad = jnp.concatenate([probs, jnp.zeros((_PAD_K - k,), jnp.float32)])
    mpad = jnp.concatenate([initial_means, jnp.zeros((32 - d,), jnp.float32)])
    spad = jnp.concatenate([initial_stds, jnp.ones((32 - d,), jnp.float32)])
    out = _qnorm(x.reshape(-1), qpad, ppad, mpad, spad)
    return out.reshape(x.shape)


# speedup vs baseline: 131.8879x; 1.5244x over previous
import functools

import jax
import jax.numpy as jnp
from jax import lax
from jax.experimental import pallas as pl
from jax.experimental.pallas import tpu as pltpu
from jax.experimental.pallas import tpu_sc as plsc

_K = 99
_PAD_K = 128
_EPS = 1e-05
_D = 26
_B = 16384
_N = _B * _D
_NW = 32
_CHUNK = _N // _NW
_PERIOD = 13
_GROUPS = _CHUNK // (16 * _PERIOD)


def _body(x_hbm, q_hbm, p_hbm, m_hbm, s_hbm, out_hbm,
          x_v, o_v, q_v, p_v, m_v, s_v, patq_v, patm_v, pats_v):
    wid = lax.axis_index("s") * 2 + lax.axis_index("c")
    base = wid * _CHUNK

    pltpu.sync_copy(x_hbm.at[pl.ds(base, _CHUNK)], x_v)
    pltpu.sync_copy(q_hbm, q_v)
    pltpu.sync_copy(p_hbm, p_v)
    pltpu.sync_copy(m_hbm, m_v)
    pltpu.sync_copy(s_hbm, s_v)

    for j in range(_PERIOD):
        lane = lax.iota(jnp.int32, 16) + (j * 16)
        dd = lane % _D
        patq_v[pl.ds(j * 16, 16)] = dd * _PAD_K
        patm_v[pl.ds(j * 16, 16)] = plsc.load_gather(m_v, [dd])
        pats_v[pl.ds(j * 16, 16)] = 1.0 / plsc.load_gather(s_v, [dd])

    @plsc.parallel_loop(0, _CHUNK // 16, step=1, unroll=8)
    def body(v):
        off = v * 16
        poff = (v % _PERIOD) * 16
        xv = x_v[pl.ds(off, 16)]
        qb = patq_v[pl.ds(poff, 16)]
        mv = patm_v[pl.ds(poff, 16)]
        iv = pats_v[pl.ds(poff, 16)]
        xn = (xv - mv) * iv

        apos = qb
        for step in (64, 32, 16, 8, 4, 2, 1):
            qv = plsc.load_gather(q_v, [apos + (step - 1)])
            apos = jnp.where(qv < xn, apos + step, apos)
        idx = apos - qb

        left = jnp.maximum(idx - 1, 0)
        right = jnp.minimum(idx, _K - 1)
        ql = plsc.load_gather(q_v, [qb + left])
        qr = plsc.load_gather(q_v, [qb + right])
        pL = plsc.load_gather(p_v, [left])
        pR = plsc.load_gather(p_v, [right])

        res = pL + (xn - ql) * (pR - pL) / (qr - ql + _EPS)
        mlow = (idx == 0) & (xn < ql)
        mhigh = (idx == _K) & (xn > qr)
        ta = jnp.where(mlow, ql - xn, xn - qr)
        e = jnp.exp(-2.0 * jnp.maximum(ta, 0.0))
        th = (1.0 - e) / (1.0 + e)
        res = jnp.where(mlow, pL - pL * th, res)
        res = jnp.where(mhigh, pR + (1.0 - pR) * th, res)
        o_v[pl.ds(off, 16)] = res
    pltpu.sync_copy(o_v, out_hbm.at[pl.ds(base, _CHUNK)])


@jax.jit
def _qnorm(xf, qpad, ppad, mpad, spad):
    mesh = plsc.VectorSubcoreMesh(core_axis_name="c", subcore_axis_name="s")
    f = pl.kernel(
        _body,
        out_type=jax.ShapeDtypeStruct((_N,), jnp.float32),
        mesh=mesh,
        compiler_params=pltpu.CompilerParams(needs_layout_passes=False),
        scratch_types=[
            pltpu.VMEM((_CHUNK,), jnp.float32),
            pltpu.VMEM((_CHUNK,), jnp.float32),
            pltpu.VMEM((_D * _PAD_K,), jnp.float32),
            pltpu.VMEM((_PAD_K,), jnp.float32),
            pltpu.VMEM((32,), jnp.float32),
            pltpu.VMEM((32,), jnp.float32),
            pltpu.VMEM((16 * _PERIOD,), jnp.int32),
            pltpu.VMEM((16 * _PERIOD,), jnp.float32),
            pltpu.VMEM((16 * _PERIOD,), jnp.float32),
        ],
    )
    return f(xf, qpad, ppad, mpad, spad)


def kernel(x, quantiles, probs, initial_means, initial_stds):
    d, k = quantiles.shape
    qpad = jnp.concatenate(
        [quantiles, jnp.full((d, _PAD_K - k), jnp.inf, jnp.float32)], axis=1
    ).reshape(-1)
    ppad = jnp.concatenate([probs, jnp.zeros((_PAD_K - k,), jnp.float32)])
    mpad = jnp.concatenate([initial_means, jnp.zeros((32 - d,), jnp.float32)])
    spad = jnp.concatenate([initial_stds, jnp.ones((32 - d,), jnp.float32)])
    out = _qnorm(x.reshape(-1), qpad, ppad, mpad, spad)
    return out.reshape(x.shape)


# speedup vs baseline: 141.8098x; 1.0752x over previous
import jax
import jax.numpy as jnp
from jax import lax
from jax.experimental import pallas as pl
from jax.experimental.pallas import tpu as pltpu
from jax.experimental.pallas import tpu_sc as plsc

_K = 99
_PAD_K = 128
_EPS = 1e-05
_D = 26
_B = 16384
_NW = 32
_ROWS = _B // _NW
_CHUNK = _ROWS * _D
_VECS = _CHUNK // 16
_PERIOD = 13


def _body(x_hbm, q_hbm, p_hbm, m_hbm, s_hbm, out_hbm,
          x_v, o_v, q_v, p_v, m_v, s_v, qpad_v,
          patr_v, patq_v, pati_v, pate_v, sem):
    wid = lax.axis_index("s") * 2 + lax.axis_index("c")
    rbase = wid * _ROWS

    xcopy = pltpu.async_copy(x_hbm.at[pl.ds(rbase, _ROWS), :], x_v, sem)
    pltpu.sync_copy(q_hbm, q_v)
    pltpu.sync_copy(p_hbm, p_v)
    pltpu.sync_copy(m_hbm, m_v)
    pltpu.sync_copy(s_hbm, s_v)

    @plsc.parallel_loop(0, _D * _PAD_K // 16, step=1, unroll=4)
    def build(j):
        flat = j * 16 + lax.iota(jnp.int32, 16)
        d = lax.shift_right_logical(flat, 7)
        c = lax.bitwise_and(flat, _PAD_K - 1)
        cc = jnp.minimum(c, _K - 1)
        qv = plsc.load_gather(q_v, [d, cc])
        sv = plsc.load_gather(s_v, [d])
        mv = plsc.load_gather(m_v, [d])
        qpad_v[pl.ds(j * 16, 16)] = jnp.where(c > _K - 1, jnp.inf, qv * sv + mv)

    for j in range(_PERIOD):
        lane = lax.iota(jnp.int32, 16) + (j * 16)
        dd = lane % _D
        sv = plsc.load_gather(s_v, [dd])
        patq_v[pl.ds(j * 16, 16)] = dd * _PAD_K
        patr_v[pl.ds(j * 16, 16)] = lax.div(lane, _D)
        pati_v[pl.ds(j * 16, 16)] = 1.0 / sv
        pate_v[pl.ds(j * 16, 16)] = _EPS * sv

    xcopy.wait()

    @plsc.parallel_loop(0, _VECS, step=1, unroll=8)
    def body(v):
        g = lax.div(v, _PERIOD)
        poff = (v - g * _PERIOD) * 16
        qb = patq_v[pl.ds(poff, 16)]
        rl = patr_v[pl.ds(poff, 16)] + g * 8
        cl = lax.shift_right_logical(qb, 7)
        iv = pati_v[pl.ds(poff, 16)]
        es = pate_v[pl.ds(poff, 16)]
        xv = plsc.load_gather(x_v, [rl, cl])

        apos = qb
        for step in (64, 32, 16, 8, 4, 2, 1):
            qv = plsc.load_gather(qpad_v, [apos + (step - 1)])
            apos = jnp.where(qv < xv, apos + step, apos)
        idx = apos - qb

        left = jnp.maximum(idx - 1, 0)
        right = jnp.minimum(idx, _K - 1)
        qls = plsc.load_gather(qpad_v, [qb + left])
        qrs = plsc.load_gather(qpad_v, [qb + right])
        pL = plsc.load_gather(p_v, [left])
        pR = plsc.load_gather(p_v, [right])

        res = pL + (xv - qls) * (pR - pL) / (qrs - qls + es)
        mlow = (idx == 0) & (xv < qls)
        mhigh = (idx == _K) & (xv > qrs)
        ta = jnp.where(mlow, qls - xv, xv - qrs) * iv
        e = jnp.exp(-2.0 * jnp.maximum(ta, 0.0))
        th = (1.0 - e) / (1.0 + e)
        res = jnp.where(mlow, pL - pL * th, res)
        res = jnp.where(mhigh, pR + (1.0 - pR) * th, res)
        plsc.store_scatter(o_v, [rl, cl], res)

    pltpu.sync_copy(o_v, out_hbm.at[pl.ds(rbase, _ROWS), :])


@jax.jit
def _qnorm(x, quantiles, probs, initial_means, initial_stds):
    mesh = plsc.VectorSubcoreMesh(core_axis_name="c", subcore_axis_name="s")
    f = pl.kernel(
        _body,
        out_type=jax.ShapeDtypeStruct((_B, _D), jnp.float32),
        mesh=mesh,
        compiler_params=pltpu.CompilerParams(
            needs_layout_passes=False, use_tc_tiling_on_sc=False),
        scratch_types=[
            pltpu.VMEM((_ROWS, _D), jnp.float32),
            pltpu.VMEM((_ROWS, _D), jnp.float32),
            pltpu.VMEM((_D, _K), jnp.float32),
            pltpu.VMEM((_K,), jnp.float32),
            pltpu.VMEM((_D,), jnp.float32),
            pltpu.VMEM((_D,), jnp.float32),
            pltpu.VMEM((_D * _PAD_K,), jnp.float32),
            pltpu.VMEM((16 * _PERIOD,), jnp.int32),
            pltpu.VMEM((16 * _PERIOD,), jnp.int32),
            pltpu.VMEM((16 * _PERIOD,), jnp.float32),
            pltpu.VMEM((16 * _PERIOD,), jnp.float32),
            pltpu.SemaphoreType.DMA,
        ],
    )
    return f(x, quantiles, probs, initial_means, initial_stds)


def kernel(x, quantiles, probs, initial_means, initial_stds):
    return _qnorm(x, quantiles, probs, initial_means, initial_stds)
